# baseline (speedup 1.0000x reference)
import functools

import jax
import jax.numpy as jnp
from jax import lax
from jax.experimental import pallas as pl
from jax.experimental.pallas import tpu as pltpu
from jax.experimental.pallas import tpu_sc as plsc

VOCAB = 100000
MAXLEN = 200
EMBED = 32
BATCH = 4096

NC = 2
NS = 16
NW = NC * NS
STILE = MAXLEN // 8
CB = BATCH // 128
SPC = 4
NCHUNK = (MAXLEN // SPC)
EG = EMBED // 8

_mesh = plsc.VectorSubcoreMesh(core_axis_name="c", subcore_axis_name="s")


@functools.partial(
    pl.kernel,
    mesh=_mesh,
    compiler_params=pltpu.CompilerParams(use_tc_tiling_on_sc=False, needs_layout_passes=False),
    out_type=jax.ShapeDtypeStruct((MAXLEN, EG, CB, 8, 128), jnp.float32),
    scratch_types=[
        pltpu.VMEM((SPC, 128), jnp.int32),
        pltpu.VMEM((SPC * 128, EMBED), jnp.float32),
        pltpu.VMEM((SPC, EG, 8, 128), jnp.float32),
        pltpu.VMEM((MAXLEN * EMBED,), jnp.float32),
        pltpu.SemaphoreType.DMA,
    ],
)
def _emb(xv_hbm, tok_hbm, pos_hbm, out_hbm, idx_v, gbuf, tbuf, pos_v, gsem):
    c = lax.axis_index("s") * NC + lax.axis_index("c")
    pltpu.sync_copy(pos_hbm, pos_v)
    iota = lax.iota(jnp.int32, 16)

    def chunk_body(k, carry):
        tr = k // 2
        h = lax.rem(k, 2)
        s0 = tr * 8 + h * SPC
        pltpu.sync_copy(xv_hbm.at[tr, c, pl.ds(h * SPC, SPC)], idx_v)
        handles = []
        for sr in range(SPC):
            handles.append(
                pltpu.async_copy(
                    tok_hbm.at[idx_v.at[sr]],
                    gbuf.at[pl.ds(sr * 128, 128)],
                    gsem,
                )
            )
        for hnd in handles:
            hnd.wait()

        def s_body(sr, _):
            s = s0 + sr

            def e_body(e, _):
                cole = jnp.full((16,), e, jnp.int32)
                posb = plsc.load_gather(pos_v, [jnp.full((16,), s * EMBED + e, jnp.int32)])
                g = e // 8
                r_e = lax.rem(e, 8)
                for lb in range(8):
                    rows = iota + (sr * 128 + lb * 16)
                    vals = plsc.load_gather(gbuf, [rows, cole])
                    tbuf[sr, g, r_e, pl.ds(lb * 16, 16)] = vals + posb
                return 0

            lax.fori_loop(0, EMBED, e_body, 0)
            pltpu.sync_copy(tbuf.at[sr], out_hbm.at[s, :, c])
            return 0

        lax.fori_loop(0, SPC, s_body, 0)
        return carry

    lax.fori_loop(0, NCHUNK, chunk_body, 0)


def kernel(x, token_table, pos_table):
    xv = (
        x.astype(jnp.int32)
        .T.reshape(STILE, 8, CB, 128)
        .transpose(0, 2, 1, 3)
    )
    out5 = _emb(xv, token_table, pos_table.reshape(-1))
    return (
        out5.transpose(2, 4, 0, 1, 3)
        .reshape(BATCH, MAXLEN, EMBED)
    )


# speedup vs baseline: 1.1623x; 1.1623x over previous
import functools

import jax
import jax.numpy as jnp
from jax import lax
from jax.experimental import pallas as pl
from jax.experimental.pallas import tpu as pltpu
from jax.experimental.pallas import tpu_sc as plsc

VOCAB = 100000
MAXLEN = 200
EMBED = 32
BATCH = 4096

NC = 2
NS = 16
NW = NC * NS
STILE = MAXLEN // 8
CB = BATCH // 128
SPC = 4
NCHUNK = MAXLEN // SPC
EG = EMBED // 8

_mesh = plsc.VectorSubcoreMesh(core_axis_name="c", subcore_axis_name="s")


@functools.partial(
    pl.kernel,
    mesh=_mesh,
    compiler_params=pltpu.CompilerParams(
        use_tc_tiling_on_sc=False, needs_layout_passes=False
    ),
    out_type=jax.ShapeDtypeStruct((MAXLEN, EG, CB, 8, 128), jnp.float32),
    scratch_types=[
        pltpu.VMEM((2, SPC, 128), jnp.int32),
        pltpu.VMEM((2, SPC * 128, EMBED), jnp.float32),
        pltpu.VMEM((2, SPC, EG, 8, 128), jnp.float32),
        pltpu.VMEM((MAXLEN * EMBED,), jnp.float32),
        pltpu.SemaphoreType.DMA,
        pltpu.SemaphoreType.DMA,
        pltpu.SemaphoreType.DMA,
        pltpu.SemaphoreType.DMA,
        pltpu.SemaphoreType.DMA,
        pltpu.SemaphoreType.DMA,
    ],
)
def _emb(
    xv_hbm, tok_hbm, pos_hbm, out_hbm,
    idx_v, gbuf, tbuf, pos_v,
    isem0, isem1, gsem0, gsem1, osem0, osem1,
):
    isem = (isem0, isem1)
    gsem = (gsem0, gsem1)
    osem = (osem0, osem1)
    c = lax.axis_index("s") * NC + lax.axis_index("c")
    pltpu.sync_copy(pos_hbm, pos_v)
    iota = lax.iota(jnp.int32, 16)

    def issue_idx(tr, h, p):
        pltpu.async_copy(
            xv_hbm.at[tr, c, pl.ds(h * SPC, SPC)], idx_v.at[p], isem[p]
        )

    def wait_idx(p):
        pltpu.make_async_copy(
            xv_hbm.at[0, 0, pl.ds(0, SPC)], idx_v.at[p], isem[p]
        ).wait()

    def issue_gathers(p):
        for sr in range(SPC):
            pltpu.async_copy(
                tok_hbm.at[idx_v.at[p, sr]],
                gbuf.at[p, pl.ds(sr * 128, 128)],
                gsem[p],
            )

    def wait_gathers(p):
        pltpu.make_async_copy(
            tok_hbm.at[pl.ds(0, SPC * 128)], gbuf.at[p], gsem[p]
        ).wait()

    def transpose_add(p, s0):
        def e_body(e, _):
            cole = jnp.full((16,), e, jnp.int32)
            g = e // 8
            r_e = lax.rem(e, 8)

            def s_inner(sr, _):
                posb = plsc.load_gather(
                    pos_v, [jnp.full((16,), (s0 + sr) * EMBED + e, jnp.int32)]
                )
                for lb in range(8):
                    rows = iota + (sr * 128 + lb * 16)
                    vals = plsc.load_gather(gbuf, [jnp.full((16,), p, jnp.int32), rows, cole])
                    tbuf[p, sr, g, r_e, pl.ds(lb * 16, 16)] = vals + posb
                return 0

            lax.fori_loop(0, SPC, s_inner, 0)
            return 0

        lax.fori_loop(0, EMBED, e_body, 0)

    def issue_out(p, s0):
        for sr in range(SPC):
            pltpu.async_copy(
                tbuf.at[p, sr], out_hbm.at[s0 + sr, :, c], osem[p]
            )

    def wait_out(p):
        for sr in range(SPC):
            pltpu.make_async_copy(
                tbuf.at[p, sr], out_hbm.at[sr, :, c], osem[p]
            ).wait()

    issue_idx(0, 0, 0)
    wait_idx(0)
    issue_gathers(0)
    issue_idx(0, 1, 1)

    def loop_body(i, carry):
        for pp in range(2):
            k = 2 * i + pp
            s0 = i * 8 + pp * SPC
            np_ = 1 - pp
            if pp == 0:
                wait_idx(np_)
                issue_gathers(np_)
            else:
                @pl.when(i < STILE - 1)
                def _():
                    wait_idx(np_)
                    issue_gathers(np_)
            if pp == 0:
                @pl.when(i >= 1)
                def _():
                    wait_out(pp)
            else:
                @pl.when(k >= 2)
                def _():
                    wait_out(pp)
            wait_gathers(pp)
            @pl.when(i < STILE - 1)
            def _():
                issue_idx(i + 1, pp, pp)
            transpose_add(pp, s0)
            issue_out(pp, s0)
        return carry

    lax.fori_loop(0, STILE, loop_body, 0)
    wait_out(0)
    wait_out(1)


def kernel(x, token_table, pos_table):
    xv = (
        x.astype(jnp.int32)
        .T.reshape(STILE, 8, CB, 128)
        .transpose(0, 2, 1, 3)
    )
    out5 = _emb(xv, token_table, pos_table.reshape(-1))
    return (
        out5.transpose(2, 4, 0, 1, 3)
        .reshape(BATCH, MAXLEN, EMBED)
    )


# speedup vs baseline: 3.5715x; 3.0729x over previous
import functools

import jax
import jax.numpy as jnp
from jax import lax
from jax.experimental import pallas as pl
from jax.experimental.pallas import tpu as pltpu
from jax.experimental.pallas import tpu_sc as plsc

VOCAB = 100000
MAXLEN = 200
EMBED = 32
BATCH = 4096

NC = 2
NS = 16
NW = NC * NS
STILE = MAXLEN // 8
CB = BATCH // 128
SPC = 4
NCHUNK = MAXLEN // SPC
EG = EMBED // 8

_mesh = plsc.VectorSubcoreMesh(core_axis_name="c", subcore_axis_name="s")


@functools.partial(
    pl.kernel,
    mesh=_mesh,
    compiler_params=pltpu.CompilerParams(
        use_tc_tiling_on_sc=False, needs_layout_passes=False
    ),
    out_type=jax.ShapeDtypeStruct((MAXLEN, EG, CB, 1024), jnp.float32),
    scratch_types=[
        pltpu.VMEM((2, SPC, 128), jnp.int32),
        pltpu.VMEM((2, SPC * 128, EMBED), jnp.float32),
        pltpu.VMEM((2, SPC * EMBED * 128), jnp.float32),
        pltpu.VMEM((MAXLEN * EMBED,), jnp.float32),
        pltpu.SemaphoreType.DMA,
        pltpu.SemaphoreType.DMA,
        pltpu.SemaphoreType.DMA,
        pltpu.SemaphoreType.DMA,
        pltpu.SemaphoreType.DMA,
        pltpu.SemaphoreType.DMA,
    ],
)
def _emb(
    xv_hbm, tok_hbm, pos_hbm, out_hbm,
    idx_v, gbuf, tbuf, pos_v,
    isem0, isem1, gsem0, gsem1, osem0, osem1,
):
    isem = (isem0, isem1)
    gsem = (gsem0, gsem1)
    osem = (osem0, osem1)
    c = lax.axis_index("s") * NC + lax.axis_index("c")
    pltpu.sync_copy(pos_hbm, pos_v)

    def issue_idx(tr, h, p):
        pltpu.async_copy(
            xv_hbm.at[tr, c, pl.ds(h * SPC, SPC)], idx_v.at[p], isem[p]
        )

    def wait_idx(p):
        pltpu.make_async_copy(
            xv_hbm.at[0, 0, pl.ds(0, SPC)], idx_v.at[p], isem[p]
        ).wait()

    def issue_gathers(p):
        for sr in range(SPC):
            pltpu.async_copy(
                tok_hbm.at[idx_v.at[p, sr]],
                gbuf.at[p, pl.ds(sr * 128, 128)],
                gsem[p],
            )

    def wait_gathers(p):
        pltpu.make_async_copy(
            tok_hbm.at[pl.ds(0, SPC * 128)], gbuf.at[p], gsem[p]
        ).wait()

    def transpose_add(p, s0):
        pconst = jnp.full((16,), p, jnp.int32)

        @plsc.parallel_loop(0, EMBED, unroll=1)
        def d_body(d):
            for lb in range(8):
                b_vec = lax.iota(jnp.int32, 16) + (lb * 16)
                e_vec = lax.bitwise_and(b_vec + d, EMBED - 1)
                sidx = (e_vec << 7) + b_vec
                for sr in range(SPC):
                    row_vec = b_vec + (sr * 128)
                    vals = plsc.load_gather(gbuf, [pconst, row_vec, e_vec])
                    posb = plsc.load_gather(
                        pos_v, [e_vec + ((s0 + sr) * EMBED)]
                    )
                    plsc.store_scatter(
                        tbuf, [pconst, sidx + (sr * EMBED * 128)], vals + posb
                    )

    def issue_out(p, s0):
        for sr in range(SPC):
            for g in range(EG):
                pltpu.async_copy(
                    tbuf.at[p, pl.ds(sr * EMBED * 128 + g * 1024, 1024)],
                    out_hbm.at[s0 + sr, g, c],
                    osem[p],
                )

    def wait_out(p):
        for sr in range(SPC):
            for g in range(EG):
                pltpu.make_async_copy(
                    tbuf.at[p, pl.ds(g * 1024, 1024)],
                    out_hbm.at[sr, g, c],
                    osem[p],
                ).wait()

    issue_idx(0, 0, 0)
    wait_idx(0)
    issue_gathers(0)
    issue_idx(0, 1, 1)

    def loop_body(i, carry):
        for pp in range(2):
            k = 2 * i + pp
            s0 = i * 8 + pp * SPC
            np_ = 1 - pp
            if pp == 0:
                wait_idx(np_)
                issue_gathers(np_)
            else:
                @pl.when(i < STILE - 1)
                def _():
                    wait_idx(np_)
                    issue_gathers(np_)
            @pl.when(i >= 1)
            def _():
                wait_out(pp)
            wait_gathers(pp)
            @pl.when(i < STILE - 1)
            def _():
                issue_idx(i + 1, pp, pp)
            transpose_add(pp, s0)
            issue_out(pp, s0)
        return carry

    lax.fori_loop(0, STILE, loop_body, 0)
    wait_out(0)
    wait_out(1)


def kernel(x, token_table, pos_table):
    xv = (
        x.astype(jnp.int32)
        .T.reshape(STILE, 8, CB, 128)
        .transpose(0, 2, 1, 3)
    )
    out4 = _emb(xv, token_table, pos_table.reshape(-1))
    return (
        out4.reshape(MAXLEN, EG, CB, 8, 128)
        .transpose(2, 4, 0, 1, 3)
        .reshape(BATCH, MAXLEN, EMBED)
    )


# speedup vs baseline: 3.9392x; 1.1029x over previous
import functools

import jax
import jax.numpy as jnp
from jax import lax
from jax.experimental import pallas as pl
from jax.experimental.pallas import tpu as pltpu
from jax.experimental.pallas import tpu_sc as plsc

VOCAB = 100000
MAXLEN = 200
EMBED = 32
BATCH = 4096

NC = 2
NS = 16
NW = NC * NS
STILE = MAXLEN // 8
CB = BATCH // 128
SPC = 4
NCHUNK = MAXLEN // SPC
EG = EMBED // 4
NB = 4

_mesh = plsc.VectorSubcoreMesh(core_axis_name="c", subcore_axis_name="s")


@functools.partial(
    pl.kernel,
    mesh=_mesh,
    compiler_params=pltpu.CompilerParams(
        use_tc_tiling_on_sc=False, needs_layout_passes=False
    ),
    out_type=jax.ShapeDtypeStruct((MAXLEN, 4, CB, 1024), jnp.float32),
    scratch_types=[
        pltpu.VMEM((NB, SPC, 128), jnp.int32),
        pltpu.VMEM((NB, SPC * 128, EMBED), jnp.float32),
        pltpu.VMEM((2, SPC * EMBED * 128), jnp.float32),
        pltpu.VMEM((MAXLEN * EMBED,), jnp.float32),
        pltpu.SemaphoreType.DMA,
        pltpu.SemaphoreType.DMA,
        pltpu.SemaphoreType.DMA,
        pltpu.SemaphoreType.DMA,
        pltpu.SemaphoreType.DMA,
        pltpu.SemaphoreType.DMA,
        pltpu.SemaphoreType.DMA,
        pltpu.SemaphoreType.DMA,
        pltpu.SemaphoreType.DMA,
        pltpu.SemaphoreType.DMA,
    ],
)
def _emb(
    xv_hbm, tok_hbm, pos_hbm, out_hbm,
    idx_v, gbuf, tbuf, pos_v,
    isem0, isem1, isem2, isem3,
    gsem0, gsem1, gsem2, gsem3,
    osem0, osem1,
):
    isem = (isem0, isem1, isem2, isem3)
    gsem = (gsem0, gsem1, gsem2, gsem3)
    osem = (osem0, osem1)
    c = lax.axis_index("s") * NC + lax.axis_index("c")
    pltpu.sync_copy(pos_hbm, pos_v)

    def issue_idx(m, q):
        pltpu.async_copy(
            xv_hbm.at[m // 2, c, pl.ds(lax.rem(m, 2) * SPC, SPC)],
            idx_v.at[q],
            isem[q],
        )

    def wait_idx(q):
        pltpu.make_async_copy(
            xv_hbm.at[0, 0, pl.ds(0, SPC)], idx_v.at[q], isem[q]
        ).wait()

    def issue_gathers(q):
        for sr in range(SPC):
            pltpu.async_copy(
                tok_hbm.at[idx_v.at[q, sr]],
                gbuf.at[q, pl.ds(sr * 128, 128)],
                gsem[q],
            )

    def wait_gathers(q):
        pltpu.make_async_copy(
            tok_hbm.at[pl.ds(0, SPC * 128)], gbuf.at[q], gsem[q]
        ).wait()

    def transpose_add(q, t, s0):
        qconst = jnp.full((16,), q, jnp.int32)
        tconst = jnp.full((16,), t, jnp.int32)

        @plsc.parallel_loop(0, EMBED, unroll=1)
        def d_body(d):
            for lb in range(8):
                b_vec = lax.iota(jnp.int32, 16) + (lb * 16)
                e_vec = lax.bitwise_and(b_vec + d, EMBED - 1)
                sidx = (e_vec << 7) + b_vec
                for sr in range(SPC):
                    row_vec = b_vec + (sr * 128)
                    vals = plsc.load_gather(gbuf, [qconst, row_vec, e_vec])
                    posb = plsc.load_gather(
                        pos_v, [e_vec + ((s0 + sr) * EMBED)]
                    )
                    plsc.store_scatter(
                        tbuf, [tconst, sidx + (sr * EMBED * 128)], vals + posb
                    )

    def issue_out(t, s0):
        for sr in range(SPC):
            for g in range(4):
                pltpu.async_copy(
                    tbuf.at[t, pl.ds(sr * EMBED * 128 + g * 1024, 1024)],
                    out_hbm.at[s0 + sr, g, c],
                    osem[t],
                )

    def wait_out(t):
        for sr in range(SPC):
            for g in range(4):
                pltpu.make_async_copy(
                    tbuf.at[t, pl.ds(g * 1024, 1024)],
                    out_hbm.at[sr, g, c],
                    osem[t],
                ).wait()

    issue_idx(0, 0)
    issue_idx(1, 1)
    issue_idx(2, 2)
    issue_idx(3, 3)
    wait_idx(0)
    issue_gathers(0)
    wait_idx(1)
    issue_gathers(1)

    def step(k, q, t):
        wait_idx((q + 2) % NB)
        issue_gathers((q + 2) % NB)
        @pl.when(k >= 2)
        def _():
            wait_out(t)
        wait_gathers(q)
        @pl.when(k < NCHUNK - 4)
        def _():
            issue_idx(k + 4, q)
        transpose_add(q, t, k * SPC)
        issue_out(t, k * SPC)

    def loop_body(i, carry):
        for qq in range(NB):
            step(i * NB + qq, qq, qq % 2)
        return carry

    lax.fori_loop(0, (NCHUNK - 2) // NB, loop_body, 0)
    for k in (NCHUNK - 2, NCHUNK - 1):
        q = k % NB
        t = k % 2
        wait_out(t)
        wait_gathers(q)
        transpose_add(q, t, k * SPC)
        issue_out(t, k * SPC)
    wait_out(0)
    wait_out(1)


def kernel(x, token_table, pos_table):
    xv = (
        x.astype(jnp.int32)
        .T.reshape(STILE, 8, CB, 128)
        .transpose(0, 2, 1, 3)
    )
    out4 = _emb(xv, token_table, pos_table.reshape(-1))
    return (
        out4.reshape(MAXLEN, 4, CB, 8, 128)
        .transpose(2, 4, 0, 1, 3)
        .reshape(BATCH, MAXLEN, EMBED)
    )
